# baseline (speedup 1.0000x reference)
import jax
import jax.numpy as jnp
from jax.experimental import pallas as pl
from jax.experimental.pallas import tpu as pltpu

D = 256
TM = 512


def _fused_rows(x_ref, wcat_ref, bin_ref, bcl_ref, wro_ref, bro_ref, o_ref):
    x = x_ref[...]
    u = jnp.dot(x, wcat_ref[...], preferred_element_type=jnp.float32)
    s = jnp.tanh(u[:, :D] + bin_ref[...]) + u[:, D:] + bcl_ref[...]
    o = jnp.dot(s.astype(jnp.bfloat16), wro_ref[...],
                preferred_element_type=jnp.float32)
    o_ref[...] = o + bro_ref[...]


def kernel(x_patient, x_se, edge_index, W_in, b_in, W_se, b_se, W_cl, b_cl,
           W_ro, b_ro, Wl_0_pse, bl_0_pse, Wr_0_pse, Wl_0_rev, bl_0_rev,
           Wr_0_rev, Wl_1_pse, bl_1_pse, Wr_1_pse, Wl_1_rev, bl_1_rev,
           Wr_1_rev):
    n_out = x_patient.shape[0] - 1
    xb = x_patient.astype(jnp.bfloat16)
    wcat = jnp.concatenate([W_in.T, W_cl.T], axis=1).astype(jnp.bfloat16)
    wro = W_ro.T.astype(jnp.bfloat16)
    grid = (pl.cdiv(n_out, TM),)
    out = pl.pallas_call(
        _fused_rows,
        grid=grid,
        in_specs=[
            pl.BlockSpec((TM, D), lambda i: (i, 0)),
            pl.BlockSpec((D, 2 * D), lambda i: (0, 0)),
            pl.BlockSpec((1, D), lambda i: (0, 0)),
            pl.BlockSpec((1, D), lambda i: (0, 0)),
            pl.BlockSpec((D, D), lambda i: (0, 0)),
            pl.BlockSpec((1, D), lambda i: (0, 0)),
        ],
        out_specs=pl.BlockSpec((TM, D), lambda i: (i, 0)),
        out_shape=jax.ShapeDtypeStruct((n_out, D), jnp.float32),
        compiler_params=pltpu.CompilerParams(
            dimension_semantics=("arbitrary",)),
    )(xb, wcat, b_in.reshape(1, D), b_cl.reshape(1, D), wro,
      b_ro.reshape(1, D))
    return out


# speedup vs baseline: 1.1890x; 1.1890x over previous
import jax
import jax.numpy as jnp
from jax.experimental import pallas as pl
from jax.experimental.pallas import tpu as pltpu

D = 256
TM = 512


def _fused_rows(x_ref, wcat_ref, bin_ref, bcl_ref, wro_ref, bro_ref, o_ref):
    x = x_ref[...].astype(jnp.bfloat16)
    u = jnp.dot(x, wcat_ref[...], preferred_element_type=jnp.float32)
    s = jnp.tanh(u[:, :D] + bin_ref[...]) + u[:, D:] + bcl_ref[...]
    o = jnp.dot(s.astype(jnp.bfloat16), wro_ref[...],
                preferred_element_type=jnp.float32)
    o_ref[...] = o + bro_ref[...]


def kernel(x_patient, x_se, edge_index, W_in, b_in, W_se, b_se, W_cl, b_cl,
           W_ro, b_ro, Wl_0_pse, bl_0_pse, Wr_0_pse, Wl_0_rev, bl_0_rev,
           Wr_0_rev, Wl_1_pse, bl_1_pse, Wr_1_pse, Wl_1_rev, bl_1_rev,
           Wr_1_rev):
    n_out = x_patient.shape[0] - 1
    wcat = jnp.concatenate([W_in.T, W_cl.T], axis=1).astype(jnp.bfloat16)
    wro = W_ro.T.astype(jnp.bfloat16)
    grid = (pl.cdiv(n_out, TM),)
    out = pl.pallas_call(
        _fused_rows,
        grid=grid,
        in_specs=[
            pl.BlockSpec((TM, D), lambda i: (i, 0)),
            pl.BlockSpec((D, 2 * D), lambda i: (0, 0)),
            pl.BlockSpec((1, D), lambda i: (0, 0)),
            pl.BlockSpec((1, D), lambda i: (0, 0)),
            pl.BlockSpec((D, D), lambda i: (0, 0)),
            pl.BlockSpec((1, D), lambda i: (0, 0)),
        ],
        out_specs=pl.BlockSpec((TM, D), lambda i: (i, 0)),
        out_shape=jax.ShapeDtypeStruct((n_out, D), jnp.float32),
        compiler_params=pltpu.CompilerParams(
            dimension_semantics=("arbitrary",)),
    )(x_patient, wcat, b_in.reshape(1, D), b_cl.reshape(1, D), wro,
      b_ro.reshape(1, D))
    return out


# speedup vs baseline: 1.4007x; 1.1781x over previous
import jax
import jax.numpy as jnp
from jax.experimental import pallas as pl
from jax.experimental.pallas import tpu as pltpu

D = 256
TM = 512


def _fused_rows(x_ref, win_ref, bin_ref, wcl_ref, bcl_ref, wro_ref, bro_ref,
                o_ref):
    x = x_ref[...].astype(jnp.bfloat16)
    dn = (((1,), (1,)), ((), ()))
    t1 = jax.lax.dot_general(x, win_ref[...].astype(jnp.bfloat16), dn,
                             preferred_element_type=jnp.float32)
    s = jnp.tanh(t1 + bin_ref[...])
    s += jax.lax.dot_general(x, wcl_ref[...].astype(jnp.bfloat16), dn,
                             preferred_element_type=jnp.float32)
    s += bcl_ref[...]
    o = jax.lax.dot_general(s.astype(jnp.bfloat16),
                            wro_ref[...].astype(jnp.bfloat16), dn,
                            preferred_element_type=jnp.float32)
    o_ref[...] = o + bro_ref[...]


def kernel(x_patient, x_se, edge_index, W_in, b_in, W_se, b_se, W_cl, b_cl,
           W_ro, b_ro, Wl_0_pse, bl_0_pse, Wr_0_pse, Wl_0_rev, bl_0_rev,
           Wr_0_rev, Wl_1_pse, bl_1_pse, Wr_1_pse, Wl_1_rev, bl_1_rev,
           Wr_1_rev):
    n_out = x_patient.shape[0] - 1
    grid = (pl.cdiv(n_out, TM),)
    wspec = pl.BlockSpec((D, D), lambda i: (0, 0))
    bspec = pl.BlockSpec((1, D), lambda i: (0, 0))
    out = pl.pallas_call(
        _fused_rows,
        grid=grid,
        in_specs=[
            pl.BlockSpec((TM, D), lambda i: (i, 0)),
            wspec, bspec, wspec, bspec, wspec, bspec,
        ],
        out_specs=pl.BlockSpec((TM, D), lambda i: (i, 0)),
        out_shape=jax.ShapeDtypeStruct((n_out, D), jnp.float32),
        compiler_params=pltpu.CompilerParams(
            dimension_semantics=("arbitrary",)),
    )(x_patient, W_in, b_in.reshape(1, D), W_cl, b_cl.reshape(1, D),
      W_ro, b_ro.reshape(1, D))
    return out


# speedup vs baseline: 1.4041x; 1.0025x over previous
import jax
import jax.numpy as jnp
from jax.experimental import pallas as pl
from jax.experimental.pallas import tpu as pltpu

D = 256
TM = 512


def _fused_rows(x_ref, win_ref, bin_ref, wcl_ref, bcl_ref, wro_ref, bro_ref,
                o_ref):
    x = x_ref[...].astype(jnp.bfloat16)
    dn = (((1,), (1,)), ((), ()))
    t1 = jax.lax.dot_general(x, win_ref[...].astype(jnp.bfloat16), dn,
                             preferred_element_type=jnp.float32)
    s = jnp.tanh(t1 + bin_ref[...])
    s += jax.lax.dot_general(x, wcl_ref[...].astype(jnp.bfloat16), dn,
                             preferred_element_type=jnp.float32)
    s += bcl_ref[...]
    o = jax.lax.dot_general(s.astype(jnp.bfloat16),
                            wro_ref[...].astype(jnp.bfloat16), dn,
                            preferred_element_type=jnp.float32)
    o_ref[...] = o + bro_ref[...]


def kernel(x_patient, x_se, edge_index, W_in, b_in, W_se, b_se, W_cl, b_cl,
           W_ro, b_ro, Wl_0_pse, bl_0_pse, Wr_0_pse, Wl_0_rev, bl_0_rev,
           Wr_0_rev, Wl_1_pse, bl_1_pse, Wr_1_pse, Wl_1_rev, bl_1_rev,
           Wr_1_rev):
    n_out = x_patient.shape[0] - 1
    grid = (pl.cdiv(n_out, TM),)
    wspec = pl.BlockSpec((D, D), lambda i: (0, 0))
    bspec = pl.BlockSpec((1, D), lambda i: (0, 0))
    out = pl.pallas_call(
        _fused_rows,
        grid=grid,
        in_specs=[
            pl.BlockSpec((TM, D), lambda i: (i, 0)),
            wspec, bspec, wspec, bspec, wspec, bspec,
        ],
        out_specs=pl.BlockSpec((TM, D), lambda i: (i, 0)),
        out_shape=jax.ShapeDtypeStruct((n_out, D), jnp.float32),
        compiler_params=pltpu.CompilerParams(
            dimension_semantics=("parallel",)),
    )(x_patient, W_in, b_in.reshape(1, D), W_cl, b_cl.reshape(1, D),
      W_ro, b_ro.reshape(1, D))
    return out
